# baseline (device time: 112619 ns/iter reference)
import jax
import jax.numpy as jnp
from jax import lax
from jax.experimental import pallas as pl
from jax.experimental.pallas import tpu as pltpu


def kernel(O, Wo):
    B, S, H, D = O.shape
    K = H * D
    N = Wo.shape[1]
    S_half = S // 2

    O2 = O.reshape(B, S, K)

    def body(o_ref, w_ref, out_ref, send_buf, recv_buf, send_sem, recv_sem):
        my_x = lax.axis_index("x")
        my_y = lax.axis_index("y")
        my_z = lax.axis_index("z")
        other_y = 1 - my_y
        neighbor = (my_x, other_y, my_z)

        barrier_sem = pltpu.get_barrier_semaphore()
        pl.semaphore_signal(
            barrier_sem, inc=1,
            device_id=neighbor, device_id_type=pl.DeviceIdType.MESH,
        )
        pl.semaphore_wait(barrier_sem, 1)

        nb = other_y * S_half
        for b in range(B):
            send_buf[b, :, :] = jnp.dot(
                o_ref[b, pl.ds(nb, S_half), :], w_ref[:, :],
                preferred_element_type=jnp.float32,
            )

        rdma = pltpu.make_async_remote_copy(
            src_ref=send_buf,
            dst_ref=recv_buf,
            send_sem=send_sem,
            recv_sem=recv_sem,
            device_id=neighbor,
            device_id_type=pl.DeviceIdType.MESH,
        )
        rdma.start()

        mine = my_y * S_half
        for b in range(B):
            out_ref[b, :, :] = jnp.dot(
                o_ref[b, pl.ds(mine, S_half), :], w_ref[:, :],
                preferred_element_type=jnp.float32,
            )

        rdma.wait()

        for b in range(B):
            out_ref[b, :, :] += recv_buf[b, :, :]

    return pl.pallas_call(
        body,
        out_shape=jax.ShapeDtypeStruct((B, S_half, N), jnp.float32),
        in_specs=[
            pl.BlockSpec(memory_space=pltpu.VMEM),
            pl.BlockSpec(memory_space=pltpu.VMEM),
        ],
        out_specs=pl.BlockSpec(memory_space=pltpu.VMEM),
        scratch_shapes=[
            pltpu.VMEM((B, S_half, N), jnp.float32),
            pltpu.VMEM((B, S_half, N), jnp.float32),
            pltpu.SemaphoreType.DMA,
            pltpu.SemaphoreType.DMA,
        ],
        compiler_params=pltpu.CompilerParams(collective_id=0),
    )(O2, Wo)


# device time: 107972 ns/iter; 1.0430x vs baseline; 1.0430x over previous
import jax
import jax.numpy as jnp
from jax import lax
from jax.experimental import pallas as pl
from jax.experimental.pallas import tpu as pltpu

CHUNKS_PER_B = 4


def kernel(O, Wo):
    B, S, H, D = O.shape
    K = H * D
    N = Wo.shape[1]
    S_half = S // 2
    S_c = S_half // CHUNKS_PER_B
    n_chunks = B * CHUNKS_PER_B

    O2 = O.reshape(B, S, K)
    chunks = [(b, j * S_c) for b in range(B) for j in range(CHUNKS_PER_B)]

    def body(o_ref, w_ref, out_ref, send_buf, recv_buf, send_sems, recv_sems):
        my_x = lax.axis_index("x")
        my_y = lax.axis_index("y")
        my_z = lax.axis_index("z")
        other_y = 1 - my_y
        neighbor = (my_x, other_y, my_z)

        barrier_sem = pltpu.get_barrier_semaphore()
        pl.semaphore_signal(
            barrier_sem, inc=1,
            device_id=neighbor, device_id_type=pl.DeviceIdType.MESH,
        )
        pl.semaphore_wait(barrier_sem, 1)

        nb = other_y * S_half
        mine = my_y * S_half

        rdmas = []
        for c, (b, s0) in enumerate(chunks):
            send_buf[b, pl.ds(s0, S_c), :] = jnp.dot(
                o_ref[b, pl.ds(nb + s0, S_c), :], w_ref[:, :],
                preferred_element_type=jnp.float32,
            )
            rdma = pltpu.make_async_remote_copy(
                src_ref=send_buf.at[b, pl.ds(s0, S_c), :],
                dst_ref=recv_buf.at[b, pl.ds(s0, S_c), :],
                send_sem=send_sems.at[c],
                recv_sem=recv_sems.at[c],
                device_id=neighbor,
                device_id_type=pl.DeviceIdType.MESH,
            )
            rdma.start()
            rdmas.append(rdma)

        for b, s0 in chunks:
            out_ref[b, pl.ds(s0, S_c), :] = jnp.dot(
                o_ref[b, pl.ds(mine + s0, S_c), :], w_ref[:, :],
                preferred_element_type=jnp.float32,
            )

        for c, (b, s0) in enumerate(chunks):
            rdmas[c].wait_recv()
            out_ref[b, pl.ds(s0, S_c), :] += recv_buf[b, pl.ds(s0, S_c), :]

        for rdma in rdmas:
            rdma.wait_send()

    return pl.pallas_call(
        body,
        out_shape=jax.ShapeDtypeStruct((B, S_half, N), jnp.float32),
        in_specs=[
            pl.BlockSpec(memory_space=pltpu.VMEM),
            pl.BlockSpec(memory_space=pltpu.VMEM),
        ],
        out_specs=pl.BlockSpec(memory_space=pltpu.VMEM),
        scratch_shapes=[
            pltpu.VMEM((B, S_half, N), jnp.float32),
            pltpu.VMEM((B, S_half, N), jnp.float32),
            pltpu.SemaphoreType.DMA((n_chunks,)),
            pltpu.SemaphoreType.DMA((n_chunks,)),
        ],
        compiler_params=pltpu.CompilerParams(collective_id=0),
    )(O2, Wo)


# device time: 107955 ns/iter; 1.0432x vs baseline; 1.0002x over previous
import jax
import jax.numpy as jnp
from jax import lax
from jax.experimental import pallas as pl
from jax.experimental.pallas import tpu as pltpu

CHUNKS_PER_B = 4
N_SLOTS = 4


def kernel(O, Wo):
    B, S, H, D = O.shape
    K = H * D
    N = Wo.shape[1]
    S_half = S // 2
    S_c = S_half // CHUNKS_PER_B
    n_chunks = B * CHUNKS_PER_B

    O2 = O.reshape(B, S, K)
    chunks = [(b, j * S_c) for b in range(B) for j in range(CHUNKS_PER_B)]

    def body(o_ref, w_ref, out_ref, o_slots, w_vmem, send_buf, recv_buf,
             o_sems, w_sem, send_sems, recv_sems):
        my_x = lax.axis_index("x")
        my_y = lax.axis_index("y")
        my_z = lax.axis_index("z")
        other_y = 1 - my_y
        neighbor = (my_x, other_y, my_z)

        w_cp = pltpu.make_async_copy(w_ref, w_vmem, w_sem)
        w_cp.start()

        nb = other_y * S_half
        mine = my_y * S_half

        jobs = [(b, nb + s0, s0) for b, s0 in chunks] + \
               [(b, mine + s0, s0) for b, s0 in chunks]
        cps = [None] * len(jobs)

        def start_copy(c):
            b, row0, _ = jobs[c]
            cp = pltpu.make_async_copy(
                o_ref.at[b, pl.ds(row0, S_c), :],
                o_slots.at[c % N_SLOTS],
                o_sems.at[c % N_SLOTS],
            )
            cp.start()
            cps[c] = cp

        for c in range(N_SLOTS):
            start_copy(c)

        barrier_sem = pltpu.get_barrier_semaphore()
        pl.semaphore_signal(
            barrier_sem, inc=1,
            device_id=neighbor, device_id_type=pl.DeviceIdType.MESH,
        )
        pl.semaphore_wait(barrier_sem, 1)

        w_cp.wait()

        rdmas = []
        for c, (b, row0, s0) in enumerate(jobs):
            cps[c].wait()
            mm = jnp.dot(
                o_slots[c % N_SLOTS], w_vmem[:, :],
                preferred_element_type=jnp.float32,
            )
            if c < n_chunks:
                send_buf[b, pl.ds(s0, S_c), :] = mm
            else:
                out_ref[b, pl.ds(s0, S_c), :] = mm
            if c + N_SLOTS < len(jobs):
                start_copy(c + N_SLOTS)
            if c < n_chunks:
                rdma = pltpu.make_async_remote_copy(
                    src_ref=send_buf.at[b, pl.ds(s0, S_c), :],
                    dst_ref=recv_buf.at[b, pl.ds(s0, S_c), :],
                    send_sem=send_sems.at[c],
                    recv_sem=recv_sems.at[c],
                    device_id=neighbor,
                    device_id_type=pl.DeviceIdType.MESH,
                )
                rdma.start()
                rdmas.append(rdma)

        for c, (b, s0) in enumerate(chunks):
            rdmas[c].wait_recv()
            out_ref[b, pl.ds(s0, S_c), :] += recv_buf[b, pl.ds(s0, S_c), :]

        for rdma in rdmas:
            rdma.wait_send()

    return pl.pallas_call(
        body,
        out_shape=jax.ShapeDtypeStruct((B, S_half, N), jnp.float32),
        in_specs=[
            pl.BlockSpec(memory_space=pl.ANY),
            pl.BlockSpec(memory_space=pl.ANY),
        ],
        out_specs=pl.BlockSpec(memory_space=pltpu.VMEM),
        scratch_shapes=[
            pltpu.VMEM((N_SLOTS, S_c, K), jnp.float32),
            pltpu.VMEM((K, N), jnp.float32),
            pltpu.VMEM((B, S_half, N), jnp.float32),
            pltpu.VMEM((B, S_half, N), jnp.float32),
            pltpu.SemaphoreType.DMA((N_SLOTS,)),
            pltpu.SemaphoreType.DMA,
            pltpu.SemaphoreType.DMA((n_chunks,)),
            pltpu.SemaphoreType.DMA((n_chunks,)),
        ],
        compiler_params=pltpu.CompilerParams(collective_id=0),
    )(O2, Wo)
